# baseline (speedup 1.0000x reference)
import jax
import jax.numpy as jnp
from jax.experimental import pallas as pl
from jax.experimental.pallas import tpu as pltpu

E = 8
D = 1024
H_DIM = 2048
MAX_A = 32
T = 8
B = 128
CB = 16
NC = 32
DC = D // NC

_INV_SQRT2 = 0.7071067811865476


def _mlp_kernel(cnt_ref, ebidx_ref, x_ref, w1_hbm, b1_ref, w2_ref,
                b2_ref, mask_ref, out_ref, xs_ref, w1_buf, sems):
    e = pl.program_id(0)
    slot = jax.lax.rem(e, 2)
    nslot = jax.lax.rem(e + 1, 2)

    @pl.when(e == 0)
    def _():
        for c in range(NC):
            pltpu.make_async_copy(
                w1_hbm.at[0, pl.ds(c * DC, DC), :],
                w1_buf.at[0, pl.ds(c * DC, DC), :],
                sems.at[0, c]).start()

    @pl.when(e + 1 < E)
    def _():
        for c in range(NC):
            pltpu.make_async_copy(
                w1_hbm.at[e + 1, pl.ds(c * DC, DC), :],
                w1_buf.at[nslot, pl.ds(c * DC, DC), :],
                sems.at[nslot, c]).start()

    for c in range(NC):
        pltpu.make_async_copy(
            w1_hbm.at[e, pl.ds(c * DC, DC), :],
            w1_buf.at[slot, pl.ds(c * DC, DC), :],
            sems.at[slot, c]).wait()

    cnt = cnt_ref[e]
    nb = (cnt + CB - 1) // CB

    def blk(k, carry):
        base = k * CB
        for i in range(CB):
            b = ebidx_ref[e, base + i]
            xs_ref[pl.ds(i * T, T), :] = x_ref[pl.ds(b * T, T), :]
        h = jnp.dot(xs_ref[...], w1_buf[slot],
                    preferred_element_type=jnp.float32) + b1_ref[0]
        h = 0.5 * h * (1.0 + jax.lax.erf(h * _INV_SQRT2))
        dec = jnp.dot(h, w2_ref[0], preferred_element_type=jnp.float32)
        dec = (dec + b2_ref[0]) * mask_ref[0]
        for i in range(CB):
            b = ebidx_ref[e, base + i]

            @pl.when(base + i < cnt)
            def _store():
                out_ref[pl.ds(b * T, T), :] = dec[i * T:(i + 1) * T, :]

        return carry

    jax.lax.fori_loop(0, nb, blk, 0)


def kernel(pred_action_latents, embodiment_ids, W1, b1, W2, b2, action_mask):
    Bn, Tn, _ = pred_action_latents.shape
    N = Bn * Tn
    x = pred_action_latents.reshape(N, D)

    ids = embodiment_ids.astype(jnp.int32)
    order = jnp.argsort(ids, stable=True).astype(jnp.int32)
    sorted_ids = ids[order]
    counts = jnp.zeros((E,), jnp.int32).at[ids].add(1)
    starts = jnp.concatenate(
        [jnp.zeros((1,), jnp.int32), jnp.cumsum(counts)[:-1].astype(jnp.int32)])
    local = jnp.arange(Bn, dtype=jnp.int32) - starts[sorted_ids]
    ebidx = jnp.zeros((E, Bn), jnp.int32).at[sorted_ids, local].set(order)

    grid_spec = pltpu.PrefetchScalarGridSpec(
        num_scalar_prefetch=2,
        grid=(E,),
        in_specs=[
            pl.BlockSpec((N, D), lambda e, cnt, eb: (0, 0)),
            pl.BlockSpec(memory_space=pl.ANY),
            pl.BlockSpec((1, 1, H_DIM), lambda e, cnt, eb: (e, 0, 0)),
            pl.BlockSpec((1, H_DIM, MAX_A), lambda e, cnt, eb: (e, 0, 0)),
            pl.BlockSpec((1, 1, MAX_A), lambda e, cnt, eb: (e, 0, 0)),
            pl.BlockSpec((1, 1, MAX_A), lambda e, cnt, eb: (e, 0, 0)),
        ],
        out_specs=pl.BlockSpec((N, MAX_A), lambda e, cnt, eb: (0, 0)),
        scratch_shapes=[
            pltpu.VMEM((CB * T, D), jnp.float32),
            pltpu.VMEM((2, D, H_DIM), jnp.float32),
            pltpu.SemaphoreType.DMA((2, NC)),
        ],
    )

    out = pl.pallas_call(
        _mlp_kernel,
        grid_spec=grid_spec,
        out_shape=jax.ShapeDtypeStruct((N, MAX_A), jnp.float32),
    )(counts, ebidx,
      x, W1, b1[:, None, :], W2, b2[:, None, :], action_mask[:, None, :])
    return out.reshape(Bn, Tn, MAX_A)


# speedup vs baseline: 1.0019x; 1.0019x over previous
import jax
import jax.numpy as jnp
from jax.experimental import pallas as pl
from jax.experimental.pallas import tpu as pltpu

E = 8
D = 1024
H_DIM = 2048
MAX_A = 32
T = 8
B = 128
CB = 16
NC = 32
DC = D // NC

_INV_SQRT2 = 0.7071067811865476


def _mlp_kernel(cnt_ref, ebidx_ref, x_ref, w1_hbm, b1_ref, w2_ref,
                b2_ref, mask_ref, out_ref, xs_ref, w1_buf, sems):
    e = pl.program_id(0)
    slot = jax.lax.rem(e, 2)
    nslot = jax.lax.rem(e + 1, 2)

    @pl.when(e == 0)
    def _():
        for c in range(NC):
            pltpu.make_async_copy(
                w1_hbm.at[0, pl.ds(c * DC, DC), :],
                w1_buf.at[0, pl.ds(c * DC, DC), :],
                sems.at[0, c]).start()

    @pl.when(e + 1 < E)
    def _():
        for c in range(NC):
            pltpu.make_async_copy(
                w1_hbm.at[e + 1, pl.ds(c * DC, DC), :],
                w1_buf.at[nslot, pl.ds(c * DC, DC), :],
                sems.at[nslot, c]).start()

    for c in range(NC):
        pltpu.make_async_copy(
            w1_hbm.at[e, pl.ds(c * DC, DC), :],
            w1_buf.at[slot, pl.ds(c * DC, DC), :],
            sems.at[slot, c]).wait()

    cnt = cnt_ref[e]
    nb = (cnt + CB - 1) // CB

    def blk(k, carry):
        base = k * CB
        for i in range(CB):
            b = ebidx_ref[e, base + i]
            xs_ref[pl.ds(i * T, T), :] = x_ref[pl.ds(b * T, T), :]
        h = jnp.dot(xs_ref[...].astype(jnp.bfloat16),
                    w1_buf[slot].astype(jnp.bfloat16),
                    preferred_element_type=jnp.float32) + b1_ref[0]
        h = 0.5 * h * (1.0 + jax.lax.erf(h * _INV_SQRT2))
        dec = jnp.dot(h.astype(jnp.bfloat16),
                      w2_ref[0].astype(jnp.bfloat16),
                      preferred_element_type=jnp.float32)
        dec = (dec + b2_ref[0]) * mask_ref[0]
        for i in range(CB):
            b = ebidx_ref[e, base + i]

            @pl.when(base + i < cnt)
            def _store():
                out_ref[pl.ds(b * T, T), :] = dec[i * T:(i + 1) * T, :]

        return carry

    jax.lax.fori_loop(0, nb, blk, 0)


def kernel(pred_action_latents, embodiment_ids, W1, b1, W2, b2, action_mask):
    Bn, Tn, _ = pred_action_latents.shape
    N = Bn * Tn
    x = pred_action_latents.reshape(N, D)

    ids = embodiment_ids.astype(jnp.int32)
    order = jnp.argsort(ids, stable=True).astype(jnp.int32)
    sorted_ids = ids[order]
    counts = jnp.zeros((E,), jnp.int32).at[ids].add(1)
    starts = jnp.concatenate(
        [jnp.zeros((1,), jnp.int32), jnp.cumsum(counts)[:-1].astype(jnp.int32)])
    local = jnp.arange(Bn, dtype=jnp.int32) - starts[sorted_ids]
    ebidx = jnp.zeros((E, Bn), jnp.int32).at[sorted_ids, local].set(order)

    grid_spec = pltpu.PrefetchScalarGridSpec(
        num_scalar_prefetch=2,
        grid=(E,),
        in_specs=[
            pl.BlockSpec((N, D), lambda e, cnt, eb: (0, 0)),
            pl.BlockSpec(memory_space=pl.ANY),
            pl.BlockSpec((1, 1, H_DIM), lambda e, cnt, eb: (e, 0, 0)),
            pl.BlockSpec((1, H_DIM, MAX_A), lambda e, cnt, eb: (e, 0, 0)),
            pl.BlockSpec((1, 1, MAX_A), lambda e, cnt, eb: (e, 0, 0)),
            pl.BlockSpec((1, 1, MAX_A), lambda e, cnt, eb: (e, 0, 0)),
        ],
        out_specs=pl.BlockSpec((N, MAX_A), lambda e, cnt, eb: (0, 0)),
        scratch_shapes=[
            pltpu.VMEM((CB * T, D), jnp.float32),
            pltpu.VMEM((2, D, H_DIM), jnp.float32),
            pltpu.SemaphoreType.DMA((2, NC)),
        ],
    )

    out = pl.pallas_call(
        _mlp_kernel,
        grid_spec=grid_spec,
        out_shape=jax.ShapeDtypeStruct((N, MAX_A), jnp.float32),
    )(counts, ebidx,
      x, W1, b1[:, None, :], W2, b2[:, None, :], action_mask[:, None, :])
    return out.reshape(Bn, Tn, MAX_A)


# speedup vs baseline: 1.0999x; 1.0978x over previous
import jax
import jax.numpy as jnp
from jax.experimental import pallas as pl
from jax.experimental.pallas import tpu as pltpu

E = 8
D = 1024
H_DIM = 2048
MAX_A = 32
T = 8
B = 128
CB = 16
NC = 32
DC = D // NC

_INV_SQRT2 = 0.7071067811865476


def _mlp_kernel(cnt_ref, ebidx_ref, x_ref, w1_hbm, b1_ref, w2_ref,
                b2_ref, mask_ref, out_ref, xs_ref, w1_buf, sems):
    e = pl.program_id(0)
    slot = jax.lax.rem(e, 2)
    nslot = jax.lax.rem(e + 1, 2)

    @pl.when(e == 0)
    def _():
        for c in range(NC):
            pltpu.make_async_copy(
                w1_hbm.at[0, pl.ds(c * DC, DC), :],
                w1_buf.at[0, pl.ds(c * DC, DC), :],
                sems.at[0, c]).start()

    @pl.when(e + 1 < E)
    def _():
        for c in range(NC):
            pltpu.make_async_copy(
                w1_hbm.at[e + 1, pl.ds(c * DC, DC), :],
                w1_buf.at[nslot, pl.ds(c * DC, DC), :],
                sems.at[nslot, c]).start()

    for c in range(NC):
        pltpu.make_async_copy(
            w1_hbm.at[e, pl.ds(c * DC, DC), :],
            w1_buf.at[slot, pl.ds(c * DC, DC), :],
            sems.at[slot, c]).wait()

    cnt = cnt_ref[e]
    nb = (cnt + CB - 1) // CB

    def blk(k, carry):
        base = k * CB
        for i in range(CB):
            b = ebidx_ref[e * B + base + i]
            xs_ref[pl.ds(i * T, T), :] = x_ref[pl.ds(b * T, T), :]
        h = jnp.dot(xs_ref[...], w1_buf[slot],
                    preferred_element_type=jnp.float32) + b1_ref[0]
        h = 0.5 * h * (1.0 + jax.lax.erf(h * _INV_SQRT2))
        dec = jnp.dot(h, w2_ref[0], preferred_element_type=jnp.float32)
        dec = (dec + b2_ref[0]) * mask_ref[0]
        for i in range(CB):
            b = ebidx_ref[e * B + base + i]

            @pl.when(base + i < cnt)
            def _store():
                out_ref[pl.ds(b * T, T), :] = dec[i * T:(i + 1) * T, :]

        return carry

    jax.lax.fori_loop(0, nb, blk, 0)


def kernel(pred_action_latents, embodiment_ids, W1, b1, W2, b2, action_mask):
    Bn, Tn, _ = pred_action_latents.shape
    N = Bn * Tn
    x = pred_action_latents.reshape(N, D)

    ids = embodiment_ids.astype(jnp.int32)
    order = jnp.argsort(ids, stable=True).astype(jnp.int32)
    sorted_ids = ids[order]
    counts = jnp.zeros((E,), jnp.int32).at[ids].add(1)
    starts = jnp.concatenate(
        [jnp.zeros((1,), jnp.int32), jnp.cumsum(counts)[:-1].astype(jnp.int32)])
    local = jnp.arange(Bn, dtype=jnp.int32) - starts[sorted_ids]
    ebidx = jnp.zeros((E * Bn,), jnp.int32).at[sorted_ids * Bn + local].set(order)

    out = pl.pallas_call(
        _mlp_kernel,
        grid=(E,),
        in_specs=[
            pl.BlockSpec(memory_space=pltpu.MemorySpace.SMEM),
            pl.BlockSpec(memory_space=pltpu.MemorySpace.SMEM),
            pl.BlockSpec((N, D), lambda e: (0, 0)),
            pl.BlockSpec(memory_space=pl.ANY),
            pl.BlockSpec((1, 1, H_DIM), lambda e: (e, 0, 0)),
            pl.BlockSpec((1, H_DIM, MAX_A), lambda e: (e, 0, 0)),
            pl.BlockSpec((1, 1, MAX_A), lambda e: (e, 0, 0)),
            pl.BlockSpec((1, 1, MAX_A), lambda e: (e, 0, 0)),
        ],
        out_specs=pl.BlockSpec((N, MAX_A), lambda e: (0, 0)),
        scratch_shapes=[
            pltpu.VMEM((CB * T, D), jnp.float32),
            pltpu.VMEM((2, D, H_DIM), jnp.float32),
            pltpu.SemaphoreType.DMA((2, NC)),
        ],
        out_shape=jax.ShapeDtypeStruct((N, MAX_A), jnp.float32),
    )(counts, ebidx, x, W1, b1[:, None, :], W2, b2[:, None, :],
      action_mask[:, None, :])
    return out.reshape(Bn, Tn, MAX_A)
